# baseline (speedup 1.0000x reference)
import numpy as np
import jax
import jax.numpy as jnp
from jax.experimental import pallas as pl
from jax.experimental.pallas import tpu as pltpu


def _upsample_matrix(n):
    g = [0.25, 0.75, 0.75, 0.25]
    a = np.zeros((2 * n, n), dtype=np.float32)
    for i in range(n):
        a[2 * i, i] = g[1]
        if i > 0:
            a[2 * i, i - 1] = g[3]
        a[2 * i + 1, i] = g[2]
        if i + 1 < n:
            a[2 * i + 1, i + 1] = g[0]
    return a


def _make_body(B, H, W, S):
    C = B // S
    G = C // 2

    def _body(x_ref, aht_ref, awd_ref, o_hbm, scratch, sems):
        base = pl.program_id(0) * (B // 2)
        for s in range(S):
            slot = s % 2
            if s >= 2:
                pltpu.make_async_copy(scratch.at[slot], scratch.at[slot],
                                      sems.at[slot]).wait()
            x = x_ref[s * C:(s + 1) * C].astype(jnp.bfloat16)
            ut = jnp.einsum("bhw,hr->bwr", x, aht_ref[...],
                            preferred_element_type=jnp.float32)
            ut2 = ut.astype(jnp.bfloat16).reshape(G, 2 * W, 2 * H)
            y = jnp.einsum("gur,uv->grv", ut2, awd_ref[...],
                           preferred_element_type=jnp.float32)
            scratch[slot, :, 0] = y[:, :, :2 * W].astype(jnp.float32)
            scratch[slot, :, 1] = y[:, :, 2 * W:].astype(jnp.float32)
            pltpu.make_async_copy(scratch.at[slot],
                                  o_hbm.at[pl.ds(base + s * G, G)],
                                  sems.at[slot]).start()
        for slot in range(min(2, S)):
            pltpu.make_async_copy(scratch.at[slot], scratch.at[slot],
                                  sems.at[slot]).wait()
    return _body


def _pick_block(nc):
    for b in (256, 128, 64, 32, 16, 8, 4, 2):
        if nc % b == 0 and nc // b >= 2:
            return b
    return nc


def kernel(x):
    N, C, H, W = x.shape
    NC = N * C
    B = _pick_block(NC)
    S = max(1, B // 64)
    x2 = x.reshape(NC, H, W)
    ah = _upsample_matrix(H)
    aw = _upsample_matrix(W)
    aht = jnp.asarray(np.ascontiguousarray(ah.T), jnp.bfloat16)
    awd = np.zeros((2 * W, 4 * W), dtype=np.float32)
    awd[:W, :2 * W] = aw.T
    awd[W:, 2 * W:] = aw.T
    awd = jnp.asarray(awd, jnp.bfloat16)
    y = pl.pallas_call(
        _make_body(B, H, W, S),
        out_shape=jax.ShapeDtypeStruct((NC // 2, 2, 2 * H, 2 * W), x.dtype),
        grid=(NC // B,),
        in_specs=[pl.BlockSpec((B, H, W), lambda i: (i, 0, 0)),
                  pl.BlockSpec((H, 2 * H), lambda i: (0, 0)),
                  pl.BlockSpec((2 * W, 4 * W), lambda i: (0, 0))],
        out_specs=pl.BlockSpec(memory_space=pl.ANY),
        scratch_shapes=[
            pltpu.VMEM((2, B // S // 2, 2, 2 * H, 2 * W), jnp.float32),
            pltpu.SemaphoreType.DMA((2,)),
        ],
        compiler_params=pltpu.CompilerParams(
            dimension_semantics=("parallel",),
            vmem_limit_bytes=60 * 1024 * 1024,
        ),
    )(x2, aht, awd)
    return y.reshape(N, C, 2 * H, 2 * W)


# speedup vs baseline: 1.1863x; 1.1863x over previous
import numpy as np
import jax
import jax.numpy as jnp
from jax.experimental import pallas as pl
from jax.experimental.pallas import tpu as pltpu


def _upsample_matrix(n):
    g = [0.25, 0.75, 0.75, 0.25]
    a = np.zeros((2 * n, n), dtype=np.float32)
    for i in range(n):
        a[2 * i, i] = g[1]
        if i > 0:
            a[2 * i, i - 1] = g[3]
        a[2 * i + 1, i] = g[2]
        if i + 1 < n:
            a[2 * i + 1, i + 1] = g[0]
    return a


def _make_body(B, H, W, S):
    C = B // S
    G = C // 2

    def _body(x_ref, aht_ref, awd_ref, o_ref):
        for s in range(S):
            x = x_ref[s * C:(s + 1) * C].astype(jnp.bfloat16)
            ut = jnp.einsum("bhw,hr->bwr", x, aht_ref[...],
                            preferred_element_type=jnp.float32)
            ut2 = ut.astype(jnp.bfloat16).reshape(G, 2 * W, 2 * H)
            y = jnp.einsum("gur,uv->grv", ut2, awd_ref[...],
                           preferred_element_type=jnp.float32)
            o_ref[s * G:(s + 1) * G, 0] = y[:, :, :2 * W].astype(o_ref.dtype)
            o_ref[s * G:(s + 1) * G, 1] = y[:, :, 2 * W:].astype(o_ref.dtype)
    return _body


def _pick_block(nc):
    for b in (256, 128, 64, 32, 16, 8, 4, 2):
        if nc % b == 0 and nc // b >= 2:
            return b
    return nc


def kernel(x):
    N, C, H, W = x.shape
    NC = N * C
    B = _pick_block(NC)
    S = max(1, B // 32)
    x2 = x.reshape(NC, H, W)
    ah = _upsample_matrix(H)
    aw = _upsample_matrix(W)
    aht = jnp.asarray(np.ascontiguousarray(ah.T), jnp.bfloat16)
    awd = np.zeros((2 * W, 4 * W), dtype=np.float32)
    awd[:W, :2 * W] = aw.T
    awd[W:, 2 * W:] = aw.T
    awd = jnp.asarray(awd, jnp.bfloat16)
    y = pl.pallas_call(
        _make_body(B, H, W, S),
        out_shape=jax.ShapeDtypeStruct((NC // 2, 2, 2 * H, 2 * W), x.dtype),
        grid=(NC // B,),
        in_specs=[pl.BlockSpec((B, H, W), lambda i: (i, 0, 0)),
                  pl.BlockSpec((H, 2 * H), lambda i: (0, 0)),
                  pl.BlockSpec((2 * W, 4 * W), lambda i: (0, 0))],
        out_specs=pl.BlockSpec((B // 2, 2, 2 * H, 2 * W),
                               lambda i: (i, 0, 0, 0)),
        compiler_params=pltpu.CompilerParams(
            dimension_semantics=("parallel",),
            vmem_limit_bytes=60 * 1024 * 1024,
        ),
    )(x2, aht, awd)
    return y.reshape(N, C, 2 * H, 2 * W)
